# baseline (speedup 1.0000x reference)
import functools

import jax
import jax.numpy as jnp
from jax import lax
from jax.experimental import pallas as pl
from jax.experimental.pallas import tpu as pltpu
from jax.experimental.pallas import tpu_sc as plsc

H = 1024
E = 8
TOP_K = 2
I = 1408
S = 2816
N = 2048
P = N * TOP_K
BT = 256
NB = 24
NPAD = NB * BT

NC = 2
NS = 16
NW = NC * NS


def _router_body(x_ref, gw_ref, wt_ref, it_ref):
    x = x_ref[...]
    gw = gw_ref[...]
    logits = lax.dot_general(
        x, gw, (((1,), (1,)), ((), ())),
        preferred_element_type=jnp.float32,
    )
    m = jnp.max(logits, axis=1, keepdims=True)
    ex = jnp.exp(logits - m)
    rw = ex / jnp.sum(ex, axis=1, keepdims=True)
    eio = lax.broadcasted_iota(jnp.int32, (N, E), 1)
    m1 = jnp.max(rw, axis=1, keepdims=True)
    i1 = jnp.min(jnp.where(rw == m1, eio, E), axis=1, keepdims=True)
    rwx = jnp.where(eio == i1, -1.0, rw)
    m2 = jnp.max(rwx, axis=1, keepdims=True)
    i2 = jnp.min(jnp.where(rwx == m2, eio, E), axis=1, keepdims=True)
    sel = (eio == i1) | (eio == i2)
    wt_ref[...] = jnp.where(sel, rw, 0.0)
    it_ref[...] = jnp.where(eio == 0, i1, jnp.where(eio == 1, i2, 0))


def _router(x2d, gate_w):
    return pl.pallas_call(
        _router_body,
        out_shape=(
            jax.ShapeDtypeStruct((N, E), jnp.float32),
            jax.ShapeDtypeStruct((N, E), jnp.int32),
        ),
    )(x2d, gate_w)


def _expert_body(sinfo_ref, x_ref, w13_ref, w2_ref, tok_ref, ws_ref, ys_ref):
    b = pl.program_id(0)

    @pl.when(b < sinfo_ref[NB])
    def _():
        tok = tok_ref[0, 0, :]
        eq = tok[:, None] == lax.broadcasted_iota(jnp.int32, (BT, N), 1)
        oh = jnp.where(eq, 1.0, 0.0)
        xb = lax.dot_general(
            oh, x_ref[...], (((1,), (0,)), ((), ())),
            preferred_element_type=jnp.float32,
        )
        w13 = w13_ref[0]
        gu = lax.dot_general(
            xb, w13, (((1,), (1,)), ((), ())),
            preferred_element_type=jnp.float32,
        )
        g = gu[:, :I]
        u = gu[:, I:]
        h = g * jax.nn.sigmoid(g) * u
        w2 = w2_ref[0]
        y = lax.dot_general(
            h, w2, (((1,), (1,)), ((), ())),
            preferred_element_type=jnp.float32,
        )
        w = ws_ref[0, 0, :]
        ys_ref[...] = y * w[:, None]


def _expert_mlp(x2d, w13_stacked, w2_stacked, tok3d, w_slot3d, sinfo):
    grid_spec = pltpu.PrefetchScalarGridSpec(
        num_scalar_prefetch=1,
        grid=(NB,),
        in_specs=[
            pl.BlockSpec((N, H), lambda b, sinfo: (0, 0)),
            pl.BlockSpec((1, 2 * I, H), lambda b, sinfo: (sinfo[b], 0, 0)),
            pl.BlockSpec((1, H, I), lambda b, sinfo: (sinfo[b], 0, 0)),
            pl.BlockSpec((1, 1, BT), lambda b, sinfo: (b, 0, 0)),
            pl.BlockSpec((1, 1, BT), lambda b, sinfo: (b, 0, 0)),
        ],
        out_specs=pl.BlockSpec((BT, H), lambda b, sinfo: (b, 0)),
    )
    return pl.pallas_call(
        _expert_body,
        grid_spec=grid_spec,
        out_shape=jax.ShapeDtypeStruct((NPAD, H), jnp.float32),
        compiler_params=pltpu.CompilerParams(
            dimension_semantics=("arbitrary",),
        ),
    )(sinfo, x2d, w13_stacked, w2_stacked, tok3d, w_slot3d)


_TB = 512


_SC2 = S // 2


def _shared_body(x_ref, wg_ref, wu_ref, wd_ref, weg_ref, out_ref):
    c = pl.program_id(1)
    xb = x_ref[...]
    g = lax.dot_general(xb, wg_ref[...], (((1,), (1,)), ((), ())),
                        preferred_element_type=jnp.float32)
    u = lax.dot_general(xb, wu_ref[...], (((1,), (1,)), ((), ())),
                        preferred_element_type=jnp.float32)
    h = g * jax.nn.sigmoid(g) * u
    y = lax.dot_general(h, wd_ref[...], (((1,), (1,)), ((), ())),
                        preferred_element_type=jnp.float32)

    @pl.when(c == 0)
    def _():
        out_ref[...] = y

    @pl.when(c == 1)
    def _():
        gate_logit = jnp.sum(x_ref[...] * weg_ref[...], axis=1,
                             keepdims=True)
        out_ref[...] = (out_ref[...] + y) * jax.nn.sigmoid(gate_logit)


def _shared_expert(x2d, wgu, wd, weg):
    return pl.pallas_call(
        _shared_body,
        grid=(N // _TB, 2),
        in_specs=[
            pl.BlockSpec((_TB, H), lambda t, c: (t, 0)),
            pl.BlockSpec((_SC2, H), lambda t, c: (c, 0)),
            pl.BlockSpec((_SC2, H), lambda t, c: (2 + c, 0)),
            pl.BlockSpec((H, _SC2), lambda t, c: (0, c)),
            pl.BlockSpec((1, H), lambda t, c: (0, 0)),
        ],
        out_specs=pl.BlockSpec((_TB, H), lambda t, c: (t, 0)),
        out_shape=jax.ShapeDtypeStruct((N, H), jnp.float32),
        compiler_params=pltpu.CompilerParams(
            dimension_semantics=("parallel", "arbitrary"),
        ),
    )(x2d, wgu, wgu, wd, weg)


_CCH = 4
_CCG = 16


def _combine_body(sh_hbm, ys_hbm, s1_hbm, s2_hbm, out_hbm,
                  i1_v, i2_v, sh0, sh1, y10, y11, y20, y21,
                  gsh0, gsh1, gy10, gy11, gy20, gy21, so0, so1):
    wid = lax.axis_index("s") * NC + lax.axis_index("c")
    base = wid * (N // NW)
    pltpu.sync_copy(s1_hbm.at[wid], i1_v)
    pltpu.sync_copy(s2_hbm.at[wid], i2_v)
    shb = [sh0, sh1]
    y1b = [y10, y11]
    y2b = [y20, y21]
    gsh = [gsh0, gsh1]
    gy1 = [gy10, gy11]
    gy2 = [gy20, gy21]
    sob = [so0, so1]

    def fire_loads(ch):
        p = ch % 2
        return (
            pltpu.async_copy(
                sh_hbm.at[pl.ds(base + ch * _CCG, _CCG)], shb[p], gsh[p]),
            pltpu.async_copy(ys_hbm.at[i1_v.at[ch]], y1b[p], gy1[p]),
            pltpu.async_copy(ys_hbm.at[i2_v.at[ch]], y2b[p], gy2[p]),
        )

    loads = {0: fire_loads(0), 1: fire_loads(1)}
    stores = {}
    for ch in range(_CCH):
        p = ch % 2
        for hdl in loads[ch]:
            hdl.wait()

        def row(r, carry):
            def col(j, carry2):
                sl = pl.ds(j * 16, 16)
                shb[p][r, sl] = shb[p][r, sl] + y1b[p][r, sl] + y2b[p][r, sl]
                return carry2
            lax.fori_loop(0, H // 16, col, 0)
            return carry
        lax.fori_loop(0, _CCG, row, 0)
        stores[ch] = pltpu.async_copy(
            shb[p], out_hbm.at[pl.ds(base + ch * _CCG, _CCG)], sob[p])
        if ch + 2 < _CCH:
            stores[ch].wait()
            loads[ch + 2] = fire_loads(ch + 2)
    stores[_CCH - 2].wait()
    stores[_CCH - 1].wait()


def _sc_combine(shared_out, ys, s1, s2):
    mesh = plsc.VectorSubcoreMesh(core_axis_name="c", subcore_axis_name="s")
    k = functools.partial(
        pl.kernel,
        out_type=jax.ShapeDtypeStruct((N, H), jnp.float32),
        mesh=mesh,
        scratch_types=[
            pltpu.VMEM((_CCH, _CCG), jnp.int32),
            pltpu.VMEM((_CCH, _CCG), jnp.int32),
        ] + [pltpu.VMEM((_CCG, H), jnp.float32)] * 6
          + [pltpu.SemaphoreType.DMA] * 8,
    )(_combine_body)
    return k(shared_out, ys,
             s1.reshape(NW, _CCH, _CCG), s2.reshape(NW, _CCH, _CCG))


def kernel(hidden_states, gate_w, w13_stacked, w2_stacked,
           shared_gate_up_w, shared_down_w, shared_expert_gate_w):
    orig_shape = hidden_states.shape
    x2d = hidden_states.reshape(N, H)

    wt, it = _router(x2d, gate_w)

    shared_out = _shared_expert(x2d, shared_gate_up_w, shared_down_w,
                                shared_expert_gate_w)

    eidx = jnp.arange(E, dtype=jnp.int32)
    it_pair = it[:, :TOP_K].reshape(-1)
    oh = (it_pair[:, None] == eidx[None, :]).astype(jnp.int32)
    pc = jnp.cumsum(oh, axis=0)
    counts = pc[-1]
    padded = ((counts + BT - 1) // BT) * BT
    pad_start = jnp.concatenate(
        [jnp.zeros((1,), padded.dtype), jnp.cumsum(padded)[:-1]])
    rank = jnp.sum(jnp.where(oh == 1, pc - 1, 0), axis=1)
    base = jnp.sum(
        jnp.where(it_pair[:, None] == eidx[None, :],
                  pad_start[None, :], 0), axis=1)
    dst = (base + rank).astype(jnp.int32)
    w1 = jnp.sum(jnp.where(it[:, :1] == eidx[None, :], wt, 0.0), axis=1)
    w2v = jnp.sum(jnp.where(it[:, 1:2] == eidx[None, :], wt, 0.0), axis=1)
    w_pair = jnp.stack([w1, w2v], axis=1).reshape(-1)
    tok_slot = jnp.zeros((NPAD,), jnp.int32).at[dst].set(
        jnp.arange(P, dtype=jnp.int32) // TOP_K,
        unique_indices=True, mode='promise_in_bounds')
    w_slot = jnp.zeros((NPAD,), jnp.float32).at[dst].set(
        w_pair, unique_indices=True, mode='promise_in_bounds')
    dst2 = dst.reshape(N, TOP_K)
    s1 = dst2[:, 0]
    s2 = dst2[:, 1]
    nb_active = (jnp.sum(padded) // BT).astype(jnp.int32)
    block_start = jnp.arange(NB, dtype=pad_start.dtype) * BT
    eid_b = (jnp.sum(
        (block_start[:, None] >= pad_start[None, :]).astype(jnp.int32),
        axis=1) - 1).astype(jnp.int32)
    sinfo = jnp.concatenate([eid_b, nb_active[None]])

    ys = _expert_mlp(x2d, w13_stacked, w2_stacked,
                     tok_slot.reshape(NB, 1, BT),
                     w_slot.reshape(NB, 1, BT), sinfo)

    out = _sc_combine(shared_out, ys, s1, s2)
    return out.reshape(orig_shape)


# speedup vs baseline: 1.0310x; 1.0310x over previous
import functools

import jax
import jax.numpy as jnp
from jax import lax
from jax.experimental import pallas as pl
from jax.experimental.pallas import tpu as pltpu
from jax.experimental.pallas import tpu_sc as plsc

H = 1024
E = 8
TOP_K = 2
I = 1408
S = 2816
N = 2048
P = N * TOP_K
BT = 256
NB = 24
NPAD = NB * BT

NC = 2
NS = 16
NW = NC * NS


def _router_body(x_ref, gw_ref, wt_ref, it_ref):
    x = x_ref[...]
    gw = gw_ref[...]
    logits = lax.dot_general(
        x, gw, (((1,), (1,)), ((), ())),
        preferred_element_type=jnp.float32,
    )
    m = jnp.max(logits, axis=1, keepdims=True)
    ex = jnp.exp(logits - m)
    rw = ex / jnp.sum(ex, axis=1, keepdims=True)
    eio = lax.broadcasted_iota(jnp.int32, (N, E), 1)
    m1 = jnp.max(rw, axis=1, keepdims=True)
    i1 = jnp.min(jnp.where(rw == m1, eio, E), axis=1, keepdims=True)
    rwx = jnp.where(eio == i1, -1.0, rw)
    m2 = jnp.max(rwx, axis=1, keepdims=True)
    i2 = jnp.min(jnp.where(rwx == m2, eio, E), axis=1, keepdims=True)
    sel = (eio == i1) | (eio == i2)
    wt_ref[...] = jnp.where(sel, rw, 0.0)
    it_ref[...] = jnp.where(eio == 0, i1, jnp.where(eio == 1, i2, 0))


def _router(x2d, gate_w):
    return pl.pallas_call(
        _router_body,
        out_shape=(
            jax.ShapeDtypeStruct((N, E), jnp.float32),
            jax.ShapeDtypeStruct((N, E), jnp.int32),
        ),
    )(x2d, gate_w)


def _expert_body(sinfo_ref, x_ref, w13_ref, w2_ref, tok_ref, ws_ref, ys_ref):
    b = pl.program_id(0)

    @pl.when(b < sinfo_ref[NB])
    def _():
        tok = tok_ref[0, 0, :]
        eq = tok[:, None] == lax.broadcasted_iota(jnp.int32, (BT, N), 1)
        oh = jnp.where(eq, 1.0, 0.0)
        xb = lax.dot_general(
            oh, x_ref[...], (((1,), (0,)), ((), ())),
            preferred_element_type=jnp.float32,
        )
        w13 = w13_ref[0]
        gu = lax.dot_general(
            xb, w13, (((1,), (1,)), ((), ())),
            preferred_element_type=jnp.float32,
        )
        g = gu[:, :I]
        u = gu[:, I:]
        h = g * jax.nn.sigmoid(g) * u
        w2 = w2_ref[0]
        y = lax.dot_general(
            h, w2, (((1,), (1,)), ((), ())),
            preferred_element_type=jnp.float32,
        )
        w = ws_ref[0, 0, :]
        ys_ref[...] = y * w[:, None]


def _expert_mlp(x2d, w13_stacked, w2_stacked, tok3d, w_slot3d, sinfo):
    grid_spec = pltpu.PrefetchScalarGridSpec(
        num_scalar_prefetch=1,
        grid=(NB,),
        in_specs=[
            pl.BlockSpec((N, H), lambda b, sinfo: (0, 0)),
            pl.BlockSpec((1, 2 * I, H), lambda b, sinfo: (sinfo[b], 0, 0)),
            pl.BlockSpec((1, H, I), lambda b, sinfo: (sinfo[b], 0, 0)),
            pl.BlockSpec((1, 1, BT), lambda b, sinfo: (b, 0, 0)),
            pl.BlockSpec((1, 1, BT), lambda b, sinfo: (b, 0, 0)),
        ],
        out_specs=pl.BlockSpec((BT, H), lambda b, sinfo: (b, 0)),
    )
    return pl.pallas_call(
        _expert_body,
        grid_spec=grid_spec,
        out_shape=jax.ShapeDtypeStruct((NPAD, H), jnp.float32),
        compiler_params=pltpu.CompilerParams(
            dimension_semantics=("arbitrary",),
        ),
    )(sinfo, x2d, w13_stacked, w2_stacked, tok3d, w_slot3d)


_TB = 512


def _shared_body(x_ref, wgu_ref, wd_ref, weg_ref, out_ref):
    xb = x_ref[...]
    wgu = wgu_ref[...]
    gu = lax.dot_general(xb, wgu, (((1,), (1,)), ((), ())),
                         preferred_element_type=jnp.float32)
    g = gu[:, :S]
    u = gu[:, S:]
    h = g * jax.nn.sigmoid(g) * u
    wd = wd_ref[...]
    y = lax.dot_general(h, wd, (((1,), (1,)), ((), ())),
                        preferred_element_type=jnp.float32)
    gate_logit = jnp.sum(x_ref[...] * weg_ref[...], axis=1, keepdims=True)
    out_ref[...] = y * jax.nn.sigmoid(gate_logit)


def _shared_expert(x2d, wgu, wd, weg):
    return pl.pallas_call(
        _shared_body,
        grid=(N // _TB,),
        in_specs=[
            pl.BlockSpec((_TB, H), lambda t: (t, 0)),
            pl.BlockSpec((2 * S, H), lambda t: (0, 0)),
            pl.BlockSpec((H, S), lambda t: (0, 0)),
            pl.BlockSpec((1, H), lambda t: (0, 0)),
        ],
        out_specs=pl.BlockSpec((_TB, H), lambda t: (t, 0)),
        out_shape=jax.ShapeDtypeStruct((N, H), jnp.float32),
        compiler_params=pltpu.CompilerParams(
            dimension_semantics=("arbitrary",),
        ),
    )(x2d, wgu, wd, weg)


_CCH = 4
_CCG = 16


def _combine_body(sh_hbm, ys_hbm, s1_hbm, s2_hbm, out_hbm,
                  i1_v, i2_v, sh0, sh1, y10, y11, y20, y21,
                  gsh0, gsh1, gy10, gy11, gy20, gy21, so0, so1):
    wid = lax.axis_index("s") * NC + lax.axis_index("c")
    base = wid * (N // NW)
    pltpu.sync_copy(s1_hbm.at[wid], i1_v)
    pltpu.sync_copy(s2_hbm.at[wid], i2_v)
    shb = [sh0, sh1]
    y1b = [y10, y11]
    y2b = [y20, y21]
    gsh = [gsh0, gsh1]
    gy1 = [gy10, gy11]
    gy2 = [gy20, gy21]
    sob = [so0, so1]

    def fire_loads(ch):
        p = ch % 2
        return (
            pltpu.async_copy(
                sh_hbm.at[pl.ds(base + ch * _CCG, _CCG)], shb[p], gsh[p]),
            pltpu.async_copy(ys_hbm.at[i1_v.at[ch]], y1b[p], gy1[p]),
            pltpu.async_copy(ys_hbm.at[i2_v.at[ch]], y2b[p], gy2[p]),
        )

    loads = {0: fire_loads(0), 1: fire_loads(1)}
    stores = {}
    for ch in range(_CCH):
        p = ch % 2
        for hdl in loads[ch]:
            hdl.wait()

        def row(r, carry):
            def col(j, carry2):
                sl = pl.ds(j * 16, 16)
                shb[p][r, sl] = shb[p][r, sl] + y1b[p][r, sl] + y2b[p][r, sl]
                return carry2
            lax.fori_loop(0, H // 16, col, 0)
            return carry
        lax.fori_loop(0, _CCG, row, 0)
        stores[ch] = pltpu.async_copy(
            shb[p], out_hbm.at[pl.ds(base + ch * _CCG, _CCG)], sob[p])
        if ch + 2 < _CCH:
            stores[ch].wait()
            loads[ch + 2] = fire_loads(ch + 2)
    stores[_CCH - 2].wait()
    stores[_CCH - 1].wait()


def _sc_combine(shared_out, ys, s1, s2):
    mesh = plsc.VectorSubcoreMesh(core_axis_name="c", subcore_axis_name="s")
    k = functools.partial(
        pl.kernel,
        out_type=jax.ShapeDtypeStruct((N, H), jnp.float32),
        mesh=mesh,
        scratch_types=[
            pltpu.VMEM((_CCH, _CCG), jnp.int32),
            pltpu.VMEM((_CCH, _CCG), jnp.int32),
        ] + [pltpu.VMEM((_CCG, H), jnp.float32)] * 6
          + [pltpu.SemaphoreType.DMA] * 8,
    )(_combine_body)
    return k(shared_out, ys,
             s1.reshape(NW, _CCH, _CCG), s2.reshape(NW, _CCH, _CCG))


def kernel(hidden_states, gate_w, w13_stacked, w2_stacked,
           shared_gate_up_w, shared_down_w, shared_expert_gate_w):
    orig_shape = hidden_states.shape
    x2d = hidden_states.reshape(N, H)

    wt, it = _router(x2d, gate_w)

    shared_out = _shared_expert(x2d, shared_gate_up_w, shared_down_w,
                                shared_expert_gate_w)

    eidx = jnp.arange(E, dtype=jnp.int32)
    it_pair = it[:, :TOP_K].reshape(-1)
    oh = (it_pair[:, None] == eidx[None, :]).astype(jnp.int32)
    pc = jnp.cumsum(oh, axis=0)
    counts = pc[-1]
    padded = ((counts + BT - 1) // BT) * BT
    pad_start = jnp.concatenate(
        [jnp.zeros((1,), padded.dtype), jnp.cumsum(padded)[:-1]])
    rank = jnp.sum(jnp.where(oh == 1, pc - 1, 0), axis=1)
    base = jnp.sum(
        jnp.where(it_pair[:, None] == eidx[None, :],
                  pad_start[None, :], 0), axis=1)
    dst = (base + rank).astype(jnp.int32)
    w1 = jnp.sum(jnp.where(it[:, :1] == eidx[None, :], wt, 0.0), axis=1)
    w2v = jnp.sum(jnp.where(it[:, 1:2] == eidx[None, :], wt, 0.0), axis=1)
    w_pair = jnp.stack([w1, w2v], axis=1).reshape(-1)
    tok_slot = jnp.zeros((NPAD,), jnp.int32).at[dst].add(
        jnp.arange(P, dtype=jnp.int32) // TOP_K,
        unique_indices=True, mode='promise_in_bounds')
    w_slot = jnp.zeros((NPAD,), jnp.float32).at[dst].add(
        w_pair, unique_indices=True, mode='promise_in_bounds')
    dst2 = dst.reshape(N, TOP_K)
    s1 = dst2[:, 0]
    s2 = dst2[:, 1]
    nb_active = (jnp.sum(padded) // BT).astype(jnp.int32)
    block_start = jnp.arange(NB, dtype=pad_start.dtype) * BT
    eid_b = (jnp.sum(
        (block_start[:, None] >= pad_start[None, :]).astype(jnp.int32),
        axis=1) - 1).astype(jnp.int32)
    sinfo = jnp.concatenate([eid_b, nb_active[None]])

    ys = _expert_mlp(x2d, w13_stacked, w2_stacked,
                     tok_slot.reshape(NB, 1, BT),
                     w_slot.reshape(NB, 1, BT), sinfo)

    out = _sc_combine(shared_out, ys, s1, s2)
    return out.reshape(orig_shape)


# speedup vs baseline: 1.0310x; 1.0000x over previous
import functools

import jax
import jax.numpy as jnp
from jax import lax
from jax.experimental import pallas as pl
from jax.experimental.pallas import tpu as pltpu
from jax.experimental.pallas import tpu_sc as plsc

H = 1024
E = 8
TOP_K = 2
I = 1408
S = 2816
N = 2048
P = N * TOP_K
BT = 256
NB = 24
NPAD = NB * BT

NC = 2
NS = 16
NW = NC * NS


def _router_body(x_ref, gw_ref, wt_ref, it_ref):
    x = x_ref[...]
    gw = gw_ref[...]
    logits = lax.dot_general(
        x, gw, (((1,), (1,)), ((), ())),
        preferred_element_type=jnp.float32,
    )
    m = jnp.max(logits, axis=1, keepdims=True)
    ex = jnp.exp(logits - m)
    rw = ex / jnp.sum(ex, axis=1, keepdims=True)
    eio = lax.broadcasted_iota(jnp.int32, (N, E), 1)
    m1 = jnp.max(rw, axis=1, keepdims=True)
    i1 = jnp.min(jnp.where(rw == m1, eio, E), axis=1, keepdims=True)
    rwx = jnp.where(eio == i1, -1.0, rw)
    m2 = jnp.max(rwx, axis=1, keepdims=True)
    i2 = jnp.min(jnp.where(rwx == m2, eio, E), axis=1, keepdims=True)
    sel = (eio == i1) | (eio == i2)
    wt_ref[...] = jnp.where(sel, rw, 0.0)
    it_ref[...] = jnp.where(eio == 0, i1, jnp.where(eio == 1, i2, 0))


def _router(x2d, gate_w):
    return pl.pallas_call(
        _router_body,
        out_shape=(
            jax.ShapeDtypeStruct((N, E), jnp.float32),
            jax.ShapeDtypeStruct((N, E), jnp.int32),
        ),
    )(x2d, gate_w)


def _expert_body(sinfo_ref, x_ref, w13_ref, w2_ref, tok_ref, ws_ref, ys_ref):
    b = pl.program_id(0)

    @pl.when(b < sinfo_ref[NB])
    def _():
        tok = tok_ref[0, 0, :]
        eq = tok[:, None] == lax.broadcasted_iota(jnp.int32, (BT, N), 1)
        oh = jnp.where(eq, 1.0, 0.0)
        xb = lax.dot_general(
            oh, x_ref[...], (((1,), (0,)), ((), ())),
            preferred_element_type=jnp.float32,
        )
        w13 = w13_ref[0]
        gu = lax.dot_general(
            xb, w13, (((1,), (1,)), ((), ())),
            preferred_element_type=jnp.float32,
        )
        g = gu[:, :I]
        u = gu[:, I:]
        h = g * jax.nn.sigmoid(g) * u
        w2 = w2_ref[0]
        y = lax.dot_general(
            h, w2, (((1,), (1,)), ((), ())),
            preferred_element_type=jnp.float32,
        )
        w = ws_ref[0, 0, :]
        ys_ref[...] = y * w[:, None]


def _expert_mlp(x2d, w13_stacked, w2_stacked, tok3d, w_slot3d, sinfo):
    grid_spec = pltpu.PrefetchScalarGridSpec(
        num_scalar_prefetch=1,
        grid=(NB,),
        in_specs=[
            pl.BlockSpec((N, H), lambda b, sinfo: (0, 0)),
            pl.BlockSpec((1, 2 * I, H), lambda b, sinfo: (sinfo[b], 0, 0)),
            pl.BlockSpec((1, H, I), lambda b, sinfo: (sinfo[b], 0, 0)),
            pl.BlockSpec((1, 1, BT), lambda b, sinfo: (b, 0, 0)),
            pl.BlockSpec((1, 1, BT), lambda b, sinfo: (b, 0, 0)),
        ],
        out_specs=pl.BlockSpec((BT, H), lambda b, sinfo: (b, 0)),
    )
    return pl.pallas_call(
        _expert_body,
        grid_spec=grid_spec,
        out_shape=jax.ShapeDtypeStruct((NPAD, H), jnp.float32),
        compiler_params=pltpu.CompilerParams(
            dimension_semantics=("arbitrary",),
        ),
    )(sinfo, x2d, w13_stacked, w2_stacked, tok3d, w_slot3d)


_TB = 512


def _shared_body(x_ref, wgu_ref, wd_ref, weg_ref, out_ref):
    xb = x_ref[...]
    wgu = wgu_ref[...]
    gu = lax.dot_general(xb, wgu, (((1,), (1,)), ((), ())),
                         preferred_element_type=jnp.float32)
    g = gu[:, :S]
    u = gu[:, S:]
    h = g * jax.nn.sigmoid(g) * u
    wd = wd_ref[...]
    y = lax.dot_general(h, wd, (((1,), (1,)), ((), ())),
                        preferred_element_type=jnp.float32)
    gate_logit = jnp.sum(x_ref[...] * weg_ref[...], axis=1, keepdims=True)
    out_ref[...] = y * jax.nn.sigmoid(gate_logit)


def _shared_expert(x2d, wgu, wd, weg):
    return pl.pallas_call(
        _shared_body,
        grid=(N // _TB,),
        in_specs=[
            pl.BlockSpec((_TB, H), lambda t: (t, 0)),
            pl.BlockSpec((2 * S, H), lambda t: (0, 0)),
            pl.BlockSpec((H, S), lambda t: (0, 0)),
            pl.BlockSpec((1, H), lambda t: (0, 0)),
        ],
        out_specs=pl.BlockSpec((_TB, H), lambda t: (t, 0)),
        out_shape=jax.ShapeDtypeStruct((N, H), jnp.float32),
        compiler_params=pltpu.CompilerParams(
            dimension_semantics=("arbitrary",),
        ),
    )(x2d, wgu, wd, weg)


_CCH = 4
_CCG = 16


def _combine_body(sh_hbm, ys_hbm, s1_hbm, s2_hbm, out_hbm,
                  i1_v, i2_v, sh0, sh1, y10, y11, y20, y21,
                  gsh0, gsh1, gy10, gy11, gy20, gy21, so0, so1):
    wid = lax.axis_index("s") * NC + lax.axis_index("c")
    base = wid * (N // NW)
    pltpu.sync_copy(s1_hbm.at[wid], i1_v)
    pltpu.sync_copy(s2_hbm.at[wid], i2_v)
    shb = [sh0, sh1]
    y1b = [y10, y11]
    y2b = [y20, y21]
    gsh = [gsh0, gsh1]
    gy1 = [gy10, gy11]
    gy2 = [gy20, gy21]
    sob = [so0, so1]

    def fire_loads(ch):
        p = ch % 2
        return (
            pltpu.async_copy(
                sh_hbm.at[pl.ds(base + ch * _CCG, _CCG)], shb[p], gsh[p]),
            pltpu.async_copy(ys_hbm.at[i1_v.at[ch]], y1b[p], gy1[p]),
            pltpu.async_copy(ys_hbm.at[i2_v.at[ch]], y2b[p], gy2[p]),
        )

    loads = {0: fire_loads(0), 1: fire_loads(1)}
    stores = {}
    for ch in range(_CCH):
        p = ch % 2
        for hdl in loads[ch]:
            hdl.wait()

        def row(r, carry):
            def col(j, carry2):
                sl = pl.ds(j * 16, 16)
                shb[p][r, sl] = shb[p][r, sl] + y1b[p][r, sl] + y2b[p][r, sl]
                return carry2
            lax.fori_loop(0, H // 16, col, 0)
            return carry
        lax.fori_loop(0, _CCG, row, 0)
        stores[ch] = pltpu.async_copy(
            shb[p], out_hbm.at[pl.ds(base + ch * _CCG, _CCG)], sob[p])
        if ch + 2 < _CCH:
            stores[ch].wait()
            loads[ch + 2] = fire_loads(ch + 2)
    stores[_CCH - 2].wait()
    stores[_CCH - 1].wait()


def _sc_combine(shared_out, ys, s1, s2):
    mesh = plsc.VectorSubcoreMesh(core_axis_name="c", subcore_axis_name="s")
    k = functools.partial(
        pl.kernel,
        out_type=jax.ShapeDtypeStruct((N, H), jnp.float32),
        mesh=mesh,
        scratch_types=[
            pltpu.VMEM((_CCH, _CCG), jnp.int32),
            pltpu.VMEM((_CCH, _CCG), jnp.int32),
        ] + [pltpu.VMEM((_CCG, H), jnp.float32)] * 6
          + [pltpu.SemaphoreType.DMA] * 8,
    )(_combine_body)
    return k(shared_out, ys,
             s1.reshape(NW, _CCH, _CCG), s2.reshape(NW, _CCH, _CCG))


def kernel(hidden_states, gate_w, w13_stacked, w2_stacked,
           shared_gate_up_w, shared_down_w, shared_expert_gate_w):
    orig_shape = hidden_states.shape
    x2d = hidden_states.reshape(N, H)

    wt, it = _router(x2d, gate_w)

    eidx = jnp.arange(E, dtype=jnp.int32)
    it_pair = it[:, :TOP_K].reshape(-1)
    oh = (it_pair[:, None] == eidx[None, :]).astype(jnp.int32)
    pc = jnp.cumsum(oh, axis=0)
    counts = pc[-1]
    padded = ((counts + BT - 1) // BT) * BT
    pad_start = jnp.concatenate(
        [jnp.zeros((1,), padded.dtype), jnp.cumsum(padded)[:-1]])
    rank = jnp.sum(jnp.where(oh == 1, pc - 1, 0), axis=1)
    base = jnp.sum(
        jnp.where(it_pair[:, None] == eidx[None, :],
                  pad_start[None, :], 0), axis=1)
    dst = (base + rank).astype(jnp.int32)
    w1 = jnp.sum(jnp.where(it[:, :1] == eidx[None, :], wt, 0.0), axis=1)
    w2v = jnp.sum(jnp.where(it[:, 1:2] == eidx[None, :], wt, 0.0), axis=1)
    w_pair = jnp.stack([w1, w2v], axis=1).reshape(-1)
    tok_slot = jnp.zeros((NPAD,), jnp.int32).at[dst].add(
        jnp.arange(P, dtype=jnp.int32) // TOP_K,
        unique_indices=True, mode='promise_in_bounds')
    w_slot = jnp.zeros((NPAD,), jnp.float32).at[dst].add(
        w_pair, unique_indices=True, mode='promise_in_bounds')
    dst2 = dst.reshape(N, TOP_K)
    s1 = dst2[:, 0]
    s2 = dst2[:, 1]
    nb_active = (jnp.sum(padded) // BT).astype(jnp.int32)
    block_start = jnp.arange(NB, dtype=pad_start.dtype) * BT
    eid_b = (jnp.sum(
        (block_start[:, None] >= pad_start[None, :]).astype(jnp.int32),
        axis=1) - 1).astype(jnp.int32)
    sinfo = jnp.concatenate([eid_b, nb_active[None]])

    shared_out = _shared_expert(x2d, shared_gate_up_w, shared_down_w,
                                shared_expert_gate_w)

    ys = _expert_mlp(x2d, w13_stacked, w2_stacked,
                     tok_slot.reshape(NB, 1, BT),
                     w_slot.reshape(NB, 1, BT), sinfo)

    out = _sc_combine(shared_out, ys, s1, s2)
    return out.reshape(orig_shape)
